# Initial kernel scaffold; baseline (speedup 1.0000x reference)
import jax
import jax.numpy as jnp
from jax.experimental import pallas as pl


def kernel(indices, W, delta):
    raise NotImplementedError("write your pallas kernel here")


# speedup vs baseline: 1.4954x; 1.4954x over previous
import functools

import jax
import jax.numpy as jnp
from jax import lax
from jax.experimental import pallas as pl
from jax.experimental.pallas import tpu as pltpu
from jax.experimental.pallas import tpu_sc as plsc

_D = 32
_N = 16384 * 20
_NW = 32
_PER_W = _N // _NW
_C = 2048
_NCH = _PER_W // _C

_mesh = plsc.VectorSubcoreMesh(core_axis_name="c", subcore_axis_name="s")


@functools.partial(
    pl.kernel,
    out_type=jax.ShapeDtypeStruct((_N, _D), jnp.float32),
    mesh=_mesh,
    scratch_types=[
        pltpu.VMEM((_C,), jnp.int32),
        pltpu.VMEM((_C, _D), jnp.float32),
        pltpu.SemaphoreType.DMA,
    ],
    compiler_params=pltpu.CompilerParams(use_tc_tiling_on_sc=False),
)
def _gather_sum(idx_hbm, w_hbm, delta_hbm, out_hbm, idx_v, rows_v, sem):
    wid = lax.axis_index("s") * 2 + lax.axis_index("c")
    base = wid * _PER_W
    for ci in range(_NCH):
        off = base + ci * _C
        pltpu.sync_copy(idx_hbm.at[pl.ds(off, _C)], idx_v)
        pltpu.async_copy(w_hbm.at[idx_v], rows_v, sem).wait()
        pltpu.async_copy(delta_hbm.at[idx_v], rows_v, sem, add=True).wait()
        pltpu.sync_copy(rows_v, out_hbm.at[pl.ds(off, _C)])


def kernel(indices, W, delta):
    idx_flat = indices.reshape(-1).astype(jnp.int32)
    out = _gather_sum(idx_flat, W, delta)
    return out.reshape(indices.shape + (_D,))


# speedup vs baseline: 1.5023x; 1.0047x over previous
import functools

import jax
import jax.numpy as jnp
from jax import lax
from jax.experimental import pallas as pl
from jax.experimental.pallas import tpu as pltpu
from jax.experimental.pallas import tpu_sc as plsc

_D = 32
_N = 16384 * 20
_NW = 32
_PER_W = _N // _NW
_C = 1024
_NCH = _PER_W // _C
_NBUF = 3

_mesh = plsc.VectorSubcoreMesh(core_axis_name="c", subcore_axis_name="s")


@functools.partial(
    pl.kernel,
    out_type=jax.ShapeDtypeStruct((_N, _D), jnp.float32),
    mesh=_mesh,
    scratch_types=(
        [pltpu.VMEM((_C,), jnp.int32) for _ in range(_NBUF)]
        + [pltpu.VMEM((_C, _D), jnp.float32) for _ in range(_NBUF)]
        + [pltpu.SemaphoreType.DMA for _ in range(4 * _NBUF)]
    ),
    compiler_params=pltpu.CompilerParams(use_tc_tiling_on_sc=False),
)
def _gather_sum(idx_hbm, w_hbm, delta_hbm, out_hbm, *refs):
    idx_v = refs[:_NBUF]
    rows = refs[_NBUF:2 * _NBUF]
    sem_i = refs[2 * _NBUF:3 * _NBUF]
    sem_w = refs[3 * _NBUF:4 * _NBUF]
    sem_d = refs[4 * _NBUF:5 * _NBUF]
    sem_s = refs[5 * _NBUF:6 * _NBUF]

    wid = lax.axis_index("s") * 2 + lax.axis_index("c")
    base = wid * _PER_W

    cp_i = [None] * _NCH
    cp_w = [None] * _NCH
    cp_d = [None] * _NCH
    cp_s = [None] * _NCH

    def issue_idx(ci):
        b = ci % _NBUF
        cp_i[ci] = pltpu.async_copy(
            idx_hbm.at[pl.ds(base + ci * _C, _C)], idx_v[b], sem_i[b])

    def issue_w(ci):
        b = ci % _NBUF
        cp_i[ci].wait()
        if ci >= _NBUF:
            cp_s[ci - _NBUF].wait()
        cp_w[ci] = pltpu.async_copy(w_hbm.at[idx_v[b]], rows[b], sem_w[b])

    def issue_d(ci):
        b = ci % _NBUF
        cp_w[ci].wait()
        cp_d[ci] = pltpu.async_copy(
            delta_hbm.at[idx_v[b]], rows[b], sem_d[b], add=True)

    def issue_s(ci):
        b = ci % _NBUF
        cp_d[ci].wait()
        cp_s[ci] = pltpu.async_copy(
            rows[b], out_hbm.at[pl.ds(base + ci * _C, _C)], sem_s[b])

    for ci in range(min(_NBUF, _NCH)):
        issue_idx(ci)
    issue_w(0)
    if _NCH > 1:
        issue_w(1)
    issue_d(0)

    for ci in range(_NCH):
        issue_s(ci)
        if ci + _NBUF < _NCH:
            issue_idx(ci + _NBUF)
        if ci + 2 < _NCH:
            issue_w(ci + 2)
        if ci + 1 < _NCH:
            issue_d(ci + 1)

    for ci in range(max(0, _NCH - _NBUF), _NCH):
        if cp_s[ci] is not None:
            cp_s[ci].wait()


def kernel(indices, W, delta):
    idx_flat = indices.reshape(-1).astype(jnp.int32)
    out = _gather_sum(idx_flat, W, delta)
    return out.reshape(indices.shape + (_D,))


# speedup vs baseline: 1.8192x; 1.2109x over previous
import functools

import jax
import jax.numpy as jnp
from jax import lax
from jax.experimental import pallas as pl
from jax.experimental.pallas import tpu as pltpu
from jax.experimental.pallas import tpu_sc as plsc

_V = 1000000
_D = 32
_N = 16384 * 20
_NW = 32
_PER_W = _N // _NW
_C = 1024
_NCH = _PER_W // _C
_NBUF = 3

_TBLK = 4096


def _sum_t_body(wt_ref, dt_ref, o_ref):
    o_ref[...] = jnp.transpose(wt_ref[...] + dt_ref[...], (1, 0))


_sum_t = pl.pallas_call(
    _sum_t_body,
    grid=((_V + _TBLK - 1) // _TBLK,),
    in_specs=[
        pl.BlockSpec((_D, _TBLK), lambda i: (0, i)),
        pl.BlockSpec((_D, _TBLK), lambda i: (0, i)),
    ],
    out_specs=pl.BlockSpec((_TBLK, _D), lambda i: (i, 0)),
    out_shape=jax.ShapeDtypeStruct((_V, _D), jnp.float32),
)

_mesh = plsc.VectorSubcoreMesh(core_axis_name="c", subcore_axis_name="s")


@functools.partial(
    pl.kernel,
    out_type=jax.ShapeDtypeStruct((_N, _D), jnp.float32),
    mesh=_mesh,
    scratch_types=(
        [pltpu.VMEM((_C,), jnp.int32) for _ in range(_NBUF)]
        + [pltpu.VMEM((_C, _D), jnp.float32) for _ in range(_NBUF)]
        + [pltpu.SemaphoreType.DMA for _ in range(3 * _NBUF)]
    ),
    compiler_params=pltpu.CompilerParams(use_tc_tiling_on_sc=False),
)
def _gather_rows(idx_hbm, s_hbm, out_hbm, *refs):
    idx_v = refs[:_NBUF]
    rows = refs[_NBUF:2 * _NBUF]
    sem_i = refs[2 * _NBUF:3 * _NBUF]
    sem_g = refs[3 * _NBUF:4 * _NBUF]
    sem_s = refs[4 * _NBUF:5 * _NBUF]

    wid = lax.axis_index("s") * 2 + lax.axis_index("c")
    base = wid * _PER_W

    cp_i = [None] * _NCH
    cp_g = [None] * _NCH
    cp_s = [None] * _NCH

    def issue_idx(ci):
        b = ci % _NBUF
        cp_i[ci] = pltpu.async_copy(
            idx_hbm.at[pl.ds(base + ci * _C, _C)], idx_v[b], sem_i[b])

    def issue_g(ci):
        b = ci % _NBUF
        cp_i[ci].wait()
        if ci >= _NBUF:
            cp_s[ci - _NBUF].wait()
        cp_g[ci] = pltpu.async_copy(s_hbm.at[idx_v[b]], rows[b], sem_g[b])

    def issue_s(ci):
        b = ci % _NBUF
        cp_g[ci].wait()
        cp_s[ci] = pltpu.async_copy(
            rows[b], out_hbm.at[pl.ds(base + ci * _C, _C)], sem_s[b])

    for ci in range(min(_NBUF, _NCH)):
        issue_idx(ci)
    issue_g(0)

    for ci in range(_NCH):
        issue_s(ci)
        if ci + _NBUF < _NCH:
            issue_idx(ci + _NBUF)
        if ci + 1 < _NCH:
            issue_g(ci + 1)

    for ci in range(max(0, _NCH - _NBUF), _NCH):
        if cp_s[ci] is not None:
            cp_s[ci].wait()


def kernel(indices, W, delta):
    idx_flat = indices.reshape(-1).astype(jnp.int32)
    s_table = _sum_t(W.T, delta.T)
    out = _gather_rows(idx_flat, s_table)
    return out.reshape(indices.shape + (_D,))


# speedup vs baseline: 2.0261x; 1.1137x over previous
import functools

import jax
import jax.numpy as jnp
from jax import lax
from jax.experimental import pallas as pl
from jax.experimental.pallas import tpu as pltpu
from jax.experimental.pallas import tpu_sc as plsc

_V = 1000000
_D = 32
_N = 16384 * 20
_NW = 32
_PER_W = _N // _NW
_C = 1024
_NCH = _PER_W // _C
_NBUF = 3

_TBLK = 16384


def _sum_t_body(wt_ref, dt_ref, o_ref):
    o_ref[...] = jnp.transpose(wt_ref[...] + dt_ref[...], (1, 0))


_sum_t = pl.pallas_call(
    _sum_t_body,
    grid=((_V + _TBLK - 1) // _TBLK,),
    in_specs=[
        pl.BlockSpec((_D, _TBLK), lambda i: (0, i)),
        pl.BlockSpec((_D, _TBLK), lambda i: (0, i)),
    ],
    out_specs=pl.BlockSpec((_TBLK, _D), lambda i: (i, 0)),
    out_shape=jax.ShapeDtypeStruct((_V, _D), jnp.float32),
)

_mesh = plsc.VectorSubcoreMesh(core_axis_name="c", subcore_axis_name="s")


@functools.partial(
    pl.kernel,
    out_type=jax.ShapeDtypeStruct((_N, _D), jnp.float32),
    mesh=_mesh,
    scratch_types=(
        [pltpu.VMEM((_C,), jnp.int32) for _ in range(_NBUF)]
        + [pltpu.VMEM((_C, _D), jnp.float32) for _ in range(_NBUF)]
        + [pltpu.SemaphoreType.DMA for _ in range(3 * _NBUF)]
    ),
    compiler_params=pltpu.CompilerParams(use_tc_tiling_on_sc=False),
)
def _gather_rows(idx_hbm, s_hbm, out_hbm, *refs):
    idx_v = refs[:_NBUF]
    rows = refs[_NBUF:2 * _NBUF]
    sem_i = refs[2 * _NBUF:3 * _NBUF]
    sem_g = refs[3 * _NBUF:4 * _NBUF]
    sem_s = refs[4 * _NBUF:5 * _NBUF]

    wid = lax.axis_index("s") * 2 + lax.axis_index("c")
    base = wid * _PER_W

    cp_i = [None] * _NCH
    cp_g = [None] * _NCH
    cp_s = [None] * _NCH

    def issue_idx(ci):
        b = ci % _NBUF
        cp_i[ci] = pltpu.async_copy(
            idx_hbm.at[pl.ds(base + ci * _C, _C)], idx_v[b], sem_i[b])

    def issue_g(ci):
        b = ci % _NBUF
        cp_i[ci].wait()
        if ci >= _NBUF:
            cp_s[ci - _NBUF].wait()
        cp_g[ci] = pltpu.async_copy(s_hbm.at[idx_v[b]], rows[b], sem_g[b])

    def issue_s(ci):
        b = ci % _NBUF
        cp_g[ci].wait()
        cp_s[ci] = pltpu.async_copy(
            rows[b], out_hbm.at[pl.ds(base + ci * _C, _C)], sem_s[b])

    for ci in range(min(_NBUF, _NCH)):
        issue_idx(ci)
    issue_g(0)

    for ci in range(_NCH):
        issue_s(ci)
        if ci + _NBUF < _NCH:
            issue_idx(ci + _NBUF)
        if ci + 1 < _NCH:
            issue_g(ci + 1)

    for ci in range(max(0, _NCH - _NBUF), _NCH):
        if cp_s[ci] is not None:
            cp_s[ci].wait()


def kernel(indices, W, delta):
    idx_flat = indices.reshape(-1).astype(jnp.int32)
    s_table = _sum_t(W.T, delta.T)
    out = _gather_rows(idx_flat, s_table)
    return out.reshape(indices.shape + (_D,))


# speedup vs baseline: 3.3643x; 1.6605x over previous
import functools

import jax
import jax.numpy as jnp
from jax import lax
from jax.experimental import pallas as pl
from jax.experimental.pallas import tpu as pltpu
from jax.experimental.pallas import tpu_sc as plsc

_V = 1000000
_D = 32
_N = 16384 * 20
_NW = 32
_PER_W = _N // _NW
_C = 1024
_NCH = _PER_W // _C
_NBUF = 3

_TBLK = 16384
_QB = _TBLK // 4
_NSTEP = (_V + _TBLK - 1) // _TBLK
_VPACK = _NSTEP * _TBLK


def _sum_t_body(wt_ref, dt_ref, o_ref):
    s = wt_ref[...] + dt_ref[...]
    for q in range(4):
        o_ref[:, q * _D:(q + 1) * _D] = jnp.transpose(
            s[:, q * _QB:(q + 1) * _QB], (1, 0))


_sum_t = pl.pallas_call(
    _sum_t_body,
    grid=(_NSTEP,),
    in_specs=[
        pl.BlockSpec((_D, _TBLK), lambda i: (0, i)),
        pl.BlockSpec((_D, _TBLK), lambda i: (0, i)),
    ],
    out_specs=pl.BlockSpec((_QB, 4 * _D), lambda i: (i, 0)),
    out_shape=jax.ShapeDtypeStruct((_VPACK // 4, 4 * _D), jnp.float32),
)

_mesh = plsc.VectorSubcoreMesh(core_axis_name="c", subcore_axis_name="s")


@functools.partial(
    pl.kernel,
    out_type=jax.ShapeDtypeStruct((_N, _D), jnp.float32),
    mesh=_mesh,
    scratch_types=(
        [pltpu.VMEM((_C,), jnp.int32) for _ in range(_NBUF)]
        + [pltpu.VMEM((_C, _D), jnp.float32) for _ in range(_NBUF)]
        + [pltpu.SemaphoreType.DMA for _ in range(3 * _NBUF)]
    ),
    compiler_params=pltpu.CompilerParams(use_tc_tiling_on_sc=False),
)
def _gather_rows(idx_hbm, s_hbm, out_hbm, *refs):
    idx_v = refs[:_NBUF]
    rows = refs[_NBUF:2 * _NBUF]
    sem_i = refs[2 * _NBUF:3 * _NBUF]
    sem_g = refs[3 * _NBUF:4 * _NBUF]
    sem_s = refs[4 * _NBUF:5 * _NBUF]

    wid = lax.axis_index("s") * 2 + lax.axis_index("c")
    base = wid * _PER_W

    cp_i = [None] * _NCH
    cp_g = [None] * _NCH
    cp_s = [None] * _NCH

    def issue_idx(ci):
        b = ci % _NBUF
        cp_i[ci] = pltpu.async_copy(
            idx_hbm.at[pl.ds(base + ci * _C, _C)], idx_v[b], sem_i[b])

    def issue_g(ci):
        b = ci % _NBUF
        cp_i[ci].wait()

        def remap(g, _):
            x = idx_v[b][pl.ds(g * 16, 16)]
            blk = lax.shift_right_logical(x, 12)
            lane = jnp.bitwise_and(x, 4095)
            j = (lax.shift_left(lax.shift_right_logical(blk, 2), 14)
                 + lax.shift_left(lane, 2)
                 + jnp.bitwise_and(blk, 3))
            idx_v[b][pl.ds(g * 16, 16)] = j
            return ()

        lax.fori_loop(0, _C // 16, remap, ())
        if ci >= _NBUF:
            cp_s[ci - _NBUF].wait()
        cp_g[ci] = pltpu.async_copy(s_hbm.at[idx_v[b]], rows[b], sem_g[b])

    def issue_s(ci):
        b = ci % _NBUF
        cp_g[ci].wait()
        cp_s[ci] = pltpu.async_copy(
            rows[b], out_hbm.at[pl.ds(base + ci * _C, _C)], sem_s[b])

    for ci in range(min(_NBUF, _NCH)):
        issue_idx(ci)
    issue_g(0)

    for ci in range(_NCH):
        issue_s(ci)
        if ci + _NBUF < _NCH:
            issue_idx(ci + _NBUF)
        if ci + 1 < _NCH:
            issue_g(ci + 1)

    for ci in range(max(0, _NCH - _NBUF), _NCH):
        if cp_s[ci] is not None:
            cp_s[ci].wait()


def kernel(indices, W, delta):
    idx_flat = indices.reshape(-1).astype(jnp.int32)
    s_packed = _sum_t(W.T, delta.T).reshape(_VPACK, _D)
    out = _gather_rows(idx_flat, s_packed)
    return out.reshape(indices.shape + (_D,))


# speedup vs baseline: 4.4296x; 1.3166x over previous
import functools

import jax
import jax.numpy as jnp
from jax import lax
from jax.experimental import pallas as pl
from jax.experimental.pallas import tpu as pltpu
from jax.experimental.pallas import tpu_sc as plsc

_V = 1000000
_D = 32
_N = 16384 * 20
_NW = 32
_PER_W = _N // _NW
_C = 1024
_NCH = _PER_W // _C
_NBUF = 3

_TBLK = 16384
_QB = _TBLK // 4
_NSTEP = (_V + _TBLK - 1) // _TBLK
_VPACK = _NSTEP * _TBLK


def _sum_t_body(wt_ref, dt_ref, o_ref):
    s = wt_ref[...] + dt_ref[...]
    s2 = jnp.concatenate(
        [s[:, q * _QB:(q + 1) * _QB] for q in range(4)], axis=0)
    o_ref[...] = jnp.transpose(s2, (1, 0))


_sum_t = pl.pallas_call(
    _sum_t_body,
    grid=(_NSTEP,),
    in_specs=[
        pl.BlockSpec((_D, _TBLK), lambda i: (0, i)),
        pl.BlockSpec((_D, _TBLK), lambda i: (0, i)),
    ],
    out_specs=pl.BlockSpec((_QB, 4 * _D), lambda i: (i, 0)),
    out_shape=jax.ShapeDtypeStruct((_VPACK // 4, 4 * _D), jnp.float32),
)

_mesh = plsc.VectorSubcoreMesh(core_axis_name="c", subcore_axis_name="s")


@functools.partial(
    pl.kernel,
    out_type=jax.ShapeDtypeStruct((_N, _D), jnp.float32),
    mesh=_mesh,
    scratch_types=(
        [pltpu.VMEM((_C,), jnp.int32) for _ in range(_NBUF)]
        + [pltpu.VMEM((_C, _D), jnp.float32) for _ in range(_NBUF)]
        + [pltpu.SemaphoreType.DMA for _ in range(3 * _NBUF)]
    ),
    compiler_params=pltpu.CompilerParams(use_tc_tiling_on_sc=False),
)
def _gather_rows(idx_hbm, s_hbm, out_hbm, *refs):
    idx_v = refs[:_NBUF]
    rows = refs[_NBUF:2 * _NBUF]
    sem_i = refs[2 * _NBUF:3 * _NBUF]
    sem_g = refs[3 * _NBUF:4 * _NBUF]
    sem_s = refs[4 * _NBUF:5 * _NBUF]

    wid = lax.axis_index("s") * 2 + lax.axis_index("c")
    base = wid * _PER_W

    cp_i = [None] * _NCH
    cp_g = [None] * _NCH
    cp_s = [None] * _NCH

    def issue_idx(ci):
        b = ci % _NBUF
        cp_i[ci] = pltpu.async_copy(
            idx_hbm.at[pl.ds(base + ci * _C, _C)], idx_v[b], sem_i[b])

    def issue_g(ci):
        b = ci % _NBUF
        cp_i[ci].wait()

        def remap(g, _):
            x = idx_v[b][pl.ds(g * 16, 16)]
            blk = lax.shift_right_logical(x, 12)
            lane = jnp.bitwise_and(x, 4095)
            j = (lax.shift_left(lax.shift_right_logical(blk, 2), 14)
                 + lax.shift_left(lane, 2)
                 + jnp.bitwise_and(blk, 3))
            idx_v[b][pl.ds(g * 16, 16)] = j
            return ()

        lax.fori_loop(0, _C // 16, remap, ())
        if ci >= _NBUF:
            cp_s[ci - _NBUF].wait()
        cp_g[ci] = pltpu.async_copy(s_hbm.at[idx_v[b]], rows[b], sem_g[b])

    def issue_s(ci):
        b = ci % _NBUF
        cp_g[ci].wait()
        cp_s[ci] = pltpu.async_copy(
            rows[b], out_hbm.at[pl.ds(base + ci * _C, _C)], sem_s[b])

    for ci in range(min(_NBUF, _NCH)):
        issue_idx(ci)
    issue_g(0)

    for ci in range(_NCH):
        issue_s(ci)
        if ci + _NBUF < _NCH:
            issue_idx(ci + _NBUF)
        if ci + 1 < _NCH:
            issue_g(ci + 1)

    for ci in range(max(0, _NCH - _NBUF), _NCH):
        if cp_s[ci] is not None:
            cp_s[ci].wait()


def kernel(indices, W, delta):
    idx_flat = indices.reshape(-1).astype(jnp.int32)
    s_packed = _sum_t(W.T, delta.T).reshape(_VPACK, _D)
    out = _gather_rows(idx_flat, s_packed)
    return out.reshape(indices.shape + (_D,))


# speedup vs baseline: 6.3234x; 1.4275x over previous
import functools

import jax
import jax.numpy as jnp
from jax import lax
from jax.experimental import pallas as pl
from jax.experimental.pallas import tpu as pltpu
from jax.experimental.pallas import tpu_sc as plsc

_V = 1000000
_D = 32
_N = 16384 * 20
_NW = 32
_PER_W = _N // _NW
_C = 1024
_NCH = _PER_W // _C
_NBUF = 3

_TBLK = 16384
_QB = _TBLK // 4
_NSTEP = (_V + _TBLK - 1) // _TBLK
_VPACK = _NSTEP * _TBLK


def _sum_t_body(wt_ref, dt_ref, o_ref):
    s = wt_ref[...] + dt_ref[...]
    s2 = jnp.concatenate(
        [s[:, q * _QB:(q + 1) * _QB] for q in range(4)], axis=0)
    o_ref[...] = jnp.transpose(s2, (1, 0))


_sum_t = pl.pallas_call(
    _sum_t_body,
    grid=(_NSTEP,),
    in_specs=[
        pl.BlockSpec((_D, _TBLK), lambda i: (0, i)),
        pl.BlockSpec((_D, _TBLK), lambda i: (0, i)),
    ],
    out_specs=pl.BlockSpec((_QB, 4 * _D), lambda i: (i, 0)),
    out_shape=jax.ShapeDtypeStruct((_VPACK // 4, 4 * _D), jnp.float32),
)

_BB = 2048


def _to_out_body(in_ref, o_ref):
    t = jnp.transpose(in_ref[...], (1, 0))
    for h in range(20):
        o_ref[h, :, :] = t[_D * h:_D * (h + 1), :]


_to_out = pl.pallas_call(
    _to_out_body,
    grid=(16384 // _BB,),
    in_specs=[pl.BlockSpec((_BB, 20 * _D), lambda i: (i, 0))],
    out_specs=pl.BlockSpec((20, _D, _BB), lambda i: (0, 0, i)),
    out_shape=jax.ShapeDtypeStruct((20, _D, 16384), jnp.float32),
)

_mesh = plsc.VectorSubcoreMesh(core_axis_name="c", subcore_axis_name="s")


@functools.partial(
    pl.kernel,
    out_type=jax.ShapeDtypeStruct((_N, _D), jnp.float32),
    mesh=_mesh,
    scratch_types=(
        [pltpu.VMEM((_C,), jnp.int32) for _ in range(_NBUF)]
        + [pltpu.VMEM((_C, _D), jnp.float32) for _ in range(_NBUF)]
        + [pltpu.SemaphoreType.DMA for _ in range(3 * _NBUF)]
    ),
    compiler_params=pltpu.CompilerParams(use_tc_tiling_on_sc=False),
)
def _gather_rows(idx_hbm, s_hbm, out_hbm, *refs):
    idx_v = refs[:_NBUF]
    rows = refs[_NBUF:2 * _NBUF]
    sem_i = refs[2 * _NBUF:3 * _NBUF]
    sem_g = refs[3 * _NBUF:4 * _NBUF]
    sem_s = refs[4 * _NBUF:5 * _NBUF]

    wid = lax.axis_index("s") * 2 + lax.axis_index("c")
    base = wid * _PER_W

    cp_i = [None] * _NCH
    cp_g = [None] * _NCH
    cp_s = [None] * _NCH

    def issue_idx(ci):
        b = ci % _NBUF
        cp_i[ci] = pltpu.async_copy(
            idx_hbm.at[pl.ds(base + ci * _C, _C)], idx_v[b], sem_i[b])

    def issue_g(ci):
        b = ci % _NBUF
        cp_i[ci].wait()

        def remap(g, _):
            x = idx_v[b][pl.ds(g * 16, 16)]
            blk = lax.shift_right_logical(x, 12)
            lane = jnp.bitwise_and(x, 4095)
            j = (lax.shift_left(lax.shift_right_logical(blk, 2), 14)
                 + lax.shift_left(lane, 2)
                 + jnp.bitwise_and(blk, 3))
            idx_v[b][pl.ds(g * 16, 16)] = j
            return ()

        lax.fori_loop(0, _C // 16, remap, ())
        if ci >= _NBUF:
            cp_s[ci - _NBUF].wait()
        cp_g[ci] = pltpu.async_copy(s_hbm.at[idx_v[b]], rows[b], sem_g[b])

    def issue_s(ci):
        b = ci % _NBUF
        cp_g[ci].wait()
        cp_s[ci] = pltpu.async_copy(
            rows[b], out_hbm.at[pl.ds(base + ci * _C, _C)], sem_s[b])

    for ci in range(min(_NBUF, _NCH)):
        issue_idx(ci)
    issue_g(0)

    for ci in range(_NCH):
        issue_s(ci)
        if ci + _NBUF < _NCH:
            issue_idx(ci + _NBUF)
        if ci + 1 < _NCH:
            issue_g(ci + 1)

    for ci in range(max(0, _NCH - _NBUF), _NCH):
        if cp_s[ci] is not None:
            cp_s[ci].wait()


def kernel(indices, W, delta):
    idx_flat = indices.reshape(-1).astype(jnp.int32)
    s_packed = _sum_t(W.T, delta.T).reshape(_VPACK, _D)
    out = _gather_rows(idx_flat, s_packed)
    y3 = _to_out(out.reshape(16384, 20 * _D))
    return jnp.transpose(y3, (2, 0, 1))


# speedup vs baseline: 6.4652x; 1.0224x over previous
import functools

import jax
import jax.numpy as jnp
from jax import lax
from jax.experimental import pallas as pl
from jax.experimental.pallas import tpu as pltpu
from jax.experimental.pallas import tpu_sc as plsc

_V = 1000000
_D = 32
_N = 16384 * 20
_NW = 32
_PER_W = _N // _NW
_C = 1024
_NCH = _PER_W // _C
_NBUF = 3

_TBLK = 32768
_QB = _TBLK // 4
_NSTEP = (_V + _TBLK - 1) // _TBLK
_VPACK = _NSTEP * _TBLK


def _sum_t_body(wt_ref, dt_ref, o_ref):
    s = wt_ref[...] + dt_ref[...]
    s2 = jnp.concatenate(
        [s[:, q * _QB:(q + 1) * _QB] for q in range(4)], axis=0)
    o_ref[...] = jnp.transpose(s2, (1, 0))


_sum_t = pl.pallas_call(
    _sum_t_body,
    grid=(_NSTEP,),
    in_specs=[
        pl.BlockSpec((_D, _TBLK), lambda i: (0, i)),
        pl.BlockSpec((_D, _TBLK), lambda i: (0, i)),
    ],
    out_specs=pl.BlockSpec((_QB, 4 * _D), lambda i: (i, 0)),
    out_shape=jax.ShapeDtypeStruct((_VPACK // 4, 4 * _D), jnp.float32),
)

_BB = 2048


def _to_out_body(in_ref, o_ref):
    t = jnp.transpose(in_ref[...], (1, 0))
    for h in range(20):
        o_ref[h, :, :] = t[_D * h:_D * (h + 1), :]


_to_out = pl.pallas_call(
    _to_out_body,
    grid=(16384 // _BB,),
    in_specs=[pl.BlockSpec((_BB, 20 * _D), lambda i: (i, 0))],
    out_specs=pl.BlockSpec((20, _D, _BB), lambda i: (0, 0, i)),
    out_shape=jax.ShapeDtypeStruct((20, _D, 16384), jnp.float32),
)

_mesh = plsc.VectorSubcoreMesh(core_axis_name="c", subcore_axis_name="s")


@functools.partial(
    pl.kernel,
    out_type=jax.ShapeDtypeStruct((_N, _D), jnp.float32),
    mesh=_mesh,
    scratch_types=(
        [pltpu.VMEM((_C,), jnp.int32) for _ in range(_NBUF)]
        + [pltpu.VMEM((_C, _D), jnp.float32) for _ in range(_NBUF)]
        + [pltpu.SemaphoreType.DMA for _ in range(3 * _NBUF)]
    ),
    compiler_params=pltpu.CompilerParams(use_tc_tiling_on_sc=False),
)
def _gather_rows(idx_hbm, s_hbm, out_hbm, *refs):
    idx_v = refs[:_NBUF]
    rows = refs[_NBUF:2 * _NBUF]
    sem_i = refs[2 * _NBUF:3 * _NBUF]
    sem_g = refs[3 * _NBUF:4 * _NBUF]
    sem_s = refs[4 * _NBUF:5 * _NBUF]

    wid = lax.axis_index("s") * 2 + lax.axis_index("c")
    base = wid * _PER_W

    cp_i = [None] * _NCH
    cp_g = [None] * _NCH
    cp_s = [None] * _NCH

    def issue_idx(ci):
        b = ci % _NBUF
        cp_i[ci] = pltpu.async_copy(
            idx_hbm.at[pl.ds(base + ci * _C, _C)], idx_v[b], sem_i[b])

    def issue_g(ci):
        b = ci % _NBUF
        cp_i[ci].wait()

        def remap(g, _):
            x = idx_v[b][pl.ds(g * 16, 16)]
            blk = lax.shift_right_logical(x, 12)
            lane = jnp.bitwise_and(x, 4095)
            j = (lax.shift_left(lax.shift_right_logical(blk, 2), 14)
                 + lax.shift_left(lane, 2)
                 + jnp.bitwise_and(blk, 3))
            idx_v[b][pl.ds(g * 16, 16)] = j
            return ()

        lax.fori_loop(0, _C // 16, remap, ())
        if ci >= _NBUF:
            cp_s[ci - _NBUF].wait()
        cp_g[ci] = pltpu.async_copy(s_hbm.at[idx_v[b]], rows[b], sem_g[b])

    def issue_s(ci):
        b = ci % _NBUF
        cp_g[ci].wait()
        cp_s[ci] = pltpu.async_copy(
            rows[b], out_hbm.at[pl.ds(base + ci * _C, _C)], sem_s[b])

    for ci in range(min(_NBUF, _NCH)):
        issue_idx(ci)
    issue_g(0)

    for ci in range(_NCH):
        issue_s(ci)
        if ci + _NBUF < _NCH:
            issue_idx(ci + _NBUF)
        if ci + 1 < _NCH:
            issue_g(ci + 1)

    for ci in range(max(0, _NCH - _NBUF), _NCH):
        if cp_s[ci] is not None:
            cp_s[ci].wait()


def kernel(indices, W, delta):
    idx_flat = indices.reshape(-1).astype(jnp.int32)
    s_packed = _sum_t(W.T, delta.T).reshape(_VPACK, _D)
    out = _gather_rows(idx_flat, s_packed)
    y3 = _to_out(out.reshape(16384, 20 * _D))
    return jnp.transpose(y3, (2, 0, 1))


# speedup vs baseline: 6.4701x; 1.0007x over previous
import functools

import jax
import jax.numpy as jnp
from jax import lax
from jax.experimental import pallas as pl
from jax.experimental.pallas import tpu as pltpu
from jax.experimental.pallas import tpu_sc as plsc

_V = 1000000
_D = 32
_N = 16384 * 20
_NW = 32
_PER_W = _N // _NW
_C = 1024
_NCH = _PER_W // _C
_NBUF = 3

_TBLK = 32768
_QB = _TBLK // 4
_NSTEP = (_V + _TBLK - 1) // _TBLK
_VPACK = _NSTEP * _TBLK
_QSH = _QB.bit_length() - 1


def _sum_t_body(wt_ref, dt_ref, o_ref):
    s = wt_ref[...] + dt_ref[...]
    s2 = jnp.concatenate(
        [s[:, q * _QB:(q + 1) * _QB] for q in range(4)], axis=0)
    o_ref[...] = jnp.transpose(s2, (1, 0))


_sum_t = pl.pallas_call(
    _sum_t_body,
    grid=(_NSTEP,),
    in_specs=[
        pl.BlockSpec((_D, _TBLK), lambda i: (0, i)),
        pl.BlockSpec((_D, _TBLK), lambda i: (0, i)),
    ],
    out_specs=pl.BlockSpec((_QB, 4 * _D), lambda i: (i, 0)),
    out_shape=jax.ShapeDtypeStruct((_VPACK // 4, 4 * _D), jnp.float32),
)

_BB = 2048


def _to_out_body(in_ref, o_ref):
    t = jnp.transpose(in_ref[...], (1, 0))
    for h in range(20):
        o_ref[h, :, :] = t[_D * h:_D * (h + 1), :]


_to_out = pl.pallas_call(
    _to_out_body,
    grid=(16384 // _BB,),
    in_specs=[pl.BlockSpec((_BB, 20 * _D), lambda i: (i, 0))],
    out_specs=pl.BlockSpec((20, _D, _BB), lambda i: (0, 0, i)),
    out_shape=jax.ShapeDtypeStruct((20, _D, 16384), jnp.float32),
)

_mesh = plsc.VectorSubcoreMesh(core_axis_name="c", subcore_axis_name="s")


@functools.partial(
    pl.kernel,
    out_type=jax.ShapeDtypeStruct((_N, _D), jnp.float32),
    mesh=_mesh,
    scratch_types=(
        [pltpu.VMEM((_C,), jnp.int32) for _ in range(_NBUF)]
        + [pltpu.VMEM((_C, _D), jnp.float32) for _ in range(_NBUF)]
        + [pltpu.SemaphoreType.DMA for _ in range(3 * _NBUF)]
    ),
    compiler_params=pltpu.CompilerParams(use_tc_tiling_on_sc=False),
)
def _gather_rows(idx_hbm, s_hbm, out_hbm, *refs):
    idx_v = refs[:_NBUF]
    rows = refs[_NBUF:2 * _NBUF]
    sem_i = refs[2 * _NBUF:3 * _NBUF]
    sem_g = refs[3 * _NBUF:4 * _NBUF]
    sem_s = refs[4 * _NBUF:5 * _NBUF]

    wid = lax.axis_index("s") * 2 + lax.axis_index("c")
    base = wid * _PER_W

    cp_i = [None] * _NCH
    cp_g = [None] * _NCH
    cp_s = [None] * _NCH

    def issue_idx(ci):
        b = ci % _NBUF
        cp_i[ci] = pltpu.async_copy(
            idx_hbm.at[pl.ds(base + ci * _C, _C)], idx_v[b], sem_i[b])

    def issue_g(ci):
        b = ci % _NBUF
        cp_i[ci].wait()

        def remap(g, _):
            x = idx_v[b][pl.ds(g * 16, 16)]
            blk = lax.shift_right_logical(x, _QSH)
            lane = jnp.bitwise_and(x, _QB - 1)
            j = (lax.shift_left(lax.shift_right_logical(blk, 2), _QSH + 2)
                 + lax.shift_left(lane, 2)
                 + jnp.bitwise_and(blk, 3))
            idx_v[b][pl.ds(g * 16, 16)] = j
            return ()

        lax.fori_loop(0, _C // 16, remap, ())
        if ci >= _NBUF:
            cp_s[ci - _NBUF].wait()
        cp_g[ci] = pltpu.async_copy(s_hbm.at[idx_v[b]], rows[b], sem_g[b])

    def issue_s(ci):
        b = ci % _NBUF
        cp_g[ci].wait()
        cp_s[ci] = pltpu.async_copy(
            rows[b], out_hbm.at[pl.ds(base + ci * _C, _C)], sem_s[b])

    for ci in range(min(_NBUF, _NCH)):
        issue_idx(ci)
    issue_g(0)

    for ci in range(_NCH):
        issue_s(ci)
        if ci + _NBUF < _NCH:
            issue_idx(ci + _NBUF)
        if ci + 1 < _NCH:
            issue_g(ci + 1)

    for ci in range(max(0, _NCH - _NBUF), _NCH):
        if cp_s[ci] is not None:
            cp_s[ci].wait()


def kernel(indices, W, delta):
    idx_flat = indices.reshape(-1).astype(jnp.int32)
    s_packed = _sum_t(W.T, delta.T).reshape(_VPACK, _D)
    out = _gather_rows(idx_flat, s_packed)
    y3 = _to_out(out.reshape(16384, 20 * _D))
    return jnp.transpose(y3, (2, 0, 1))
